# Initial kernel scaffold; baseline (speedup 1.0000x reference)
import jax
import jax.numpy as jnp
from jax.experimental import pallas as pl


def kernel(inputs, labels, num_sampled, input_embed, out_embed, noise_idx):
    raise NotImplementedError("write your pallas kernel here")


# speedup vs baseline: 1.0625x; 1.0625x over previous
import dataclasses
import functools

import jax
import jax.numpy as jnp
from jax import lax
from jax.experimental import pallas as pl
from jax.experimental.pallas import tpu as pltpu
from jax.experimental.pallas import tpu_sc as plsc

VOCAB = 1000000
DIM = 32
BATCH = 1024
WINDOW = 20
NSAMP = 20

NCORES = 2
NSUB = 16
LANES = 16
NWORK = NCORES * NSUB
PAIRS = BATCH * WINDOW
PW = PAIRS // NWORK
CP = 64
NCHUNK = PW // CP
BPW = BATCH // NWORK
COLS = 24


def _sc_compiler_params():
    cp = pltpu.CompilerParams()
    if "needs_layout_passes" in pltpu.CompilerParams.__dataclass_fields__:
        cp = dataclasses.replace(cp, needs_layout_passes=False)
    if "use_tc_tiling_on_sc" in pltpu.CompilerParams.__dataclass_fields__:
        cp = dataclasses.replace(cp, use_tc_tiling_on_sc=False)
    return cp


def _sc_scores(inputs_f, labels_f, noise_f, input_embed, out_embed):
    mesh = plsc.VectorSubcoreMesh(core_axis_name="c", subcore_axis_name="s")

    @functools.partial(
        pl.kernel,
        compiler_params=_sc_compiler_params(),
        out_type=jax.ShapeDtypeStruct((PAIRS * COLS,), jnp.float32),
        mesh=mesh,
        scratch_types=[
            pltpu.VMEM((BPW,), jnp.int32),
            pltpu.VMEM((BPW, DIM), jnp.float32),
            pltpu.VMEM((CP,), jnp.int32),
            pltpu.VMEM((CP,), jnp.int32),
            pltpu.VMEM((CP, DIM), jnp.float32),
            pltpu.VMEM((CP, DIM), jnp.float32),
            pltpu.VMEM((CP * NSAMP,), jnp.int32),
            pltpu.VMEM((CP * NSAMP,), jnp.int32),
            pltpu.VMEM((NSAMP, CP), jnp.int32),
            pltpu.VMEM((NSAMP, CP), jnp.int32),
            pltpu.VMEM((NSAMP, CP, DIM), jnp.float32),
            pltpu.VMEM((NSAMP, CP, DIM), jnp.float32),
            pltpu.VMEM((CP * COLS,), jnp.float32),
            pltpu.VMEM((CP * COLS,), jnp.float32),
            pltpu.SemaphoreType.DMA,
            pltpu.SemaphoreType.DMA,
            pltpu.SemaphoreType.DMA,
        ],
    )
    def kern(inputs_hbm, labels_hbm, noise_hbm, iemb_hbm, oemb_hbm, scores_hbm,
             binp_idx, inp_rows, lab0, lab1, out0, out1, nlin0, nlin1,
             nt0, nt1, nr0, nr1, sc0, sc1, sem0, sem1, semi):
        lab = (lab0, lab1)
        outr = (out0, out1)
        nlin = (nlin0, nlin1)
        nt = (nt0, nt1)
        nrows = (nr0, nr1)
        scv = (sc0, sc1)
        sems = (sem0, sem1)

        wid = lax.axis_index("s") * NCORES + lax.axis_index("c")
        wp0 = wid * PW
        iota = lax.iota(jnp.int32, LANES)

        pltpu.sync_copy(inputs_hbm.at[pl.ds(wid * BPW, BPW)], binp_idx)
        pltpu.async_copy(iemb_hbm.at[binp_idx], inp_rows, semi).wait()

        def stage(c, bi):
            bp = wp0 + c * CP
            pltpu.sync_copy(labels_hbm.at[pl.ds(bp, CP)], lab[bi])
            pltpu.sync_copy(noise_hbm.at[pl.ds(bp * NSAMP, CP * NSAMP)], nlin[bi])
            for g in range(CP // LANES):
                rowbase = (iota + g * LANES) * NSAMP
                for s in range(NSAMP):
                    v = plsc.load_gather(nlin[bi], [rowbase + s])
                    nt[bi][s, pl.ds(g * LANES, LANES)] = v
            descs = [pltpu.async_copy(oemb_hbm.at[lab[bi]], outr[bi], sems[bi])]
            for s in range(NSAMP):
                descs.append(
                    pltpu.async_copy(oemb_hbm.at[nt[bi].at[s]],
                                     nrows[bi].at[s], sems[bi]))
            return descs

        def compute(c, bi):
            def group(g, carry):
                pch = iota + g * LANES
                bloc = (pch + c * CP) // WINDOW

                def dbody(d, accs):
                    dv = jnp.full((LANES,), 0, jnp.int32) + d
                    inpv = plsc.load_gather(inp_rows, [bloc, dv])
                    outv = plsc.load_gather(outr[bi], [pch, dv])
                    new = [accs[0] + inpv * outv]
                    for s in range(NSAMP):
                        sv = jnp.full((LANES,), s, jnp.int32)
                        nv = plsc.load_gather(nrows[bi], [sv, pch, dv])
                        new.append(accs[s + 1] - inpv * nv)
                    return tuple(new)

                accs0 = tuple(jnp.zeros((LANES,), jnp.float32)
                              for _ in range(NSAMP + 1))
                accs = lax.fori_loop(0, DIM, dbody, accs0)
                base = pch * COLS
                plsc.store_scatter(scv[bi], [base + NSAMP], accs[0])
                for s in range(NSAMP):
                    plsc.store_scatter(scv[bi], [base + s], accs[s + 1])
                zero = jnp.zeros((LANES,), jnp.float32)
                for pcol in range(NSAMP + 1, COLS):
                    plsc.store_scatter(scv[bi], [base + pcol], zero)
                return carry

            lax.fori_loop(0, CP // LANES, group, 0)

        pending = stage(0, 0)
        for c in range(NCHUNK):
            nxt = stage(c + 1, (c + 1) % 2) if c + 1 < NCHUNK else None
            for dsc in pending:
                dsc.wait()
            compute(c, c % 2)
            pltpu.sync_copy(scv[c % 2],
                            scores_hbm.at[pl.ds((wp0 + c * CP) * COLS,
                                                CP * COLS)])
            pending = nxt

    return kern(inputs_f, labels_f, noise_f, input_embed, out_embed)


def _tc_loss(scores):
    rows = PAIRS * COLS // 128
    x2 = scores.reshape(rows, 128)

    def body(s_ref, o_ref):
        x = s_ref[...]
        r = lax.broadcasted_iota(jnp.int32, x.shape, 0)
        cc = lax.broadcasted_iota(jnp.int32, x.shape, 1)
        j = (r * 128 + cc) % COLS
        val = jnp.where(j <= NSAMP, jnp.log(jax.nn.sigmoid(x)), 0.0)
        o_ref[0, 0] = -jnp.sum(val) / BATCH

    out = pl.pallas_call(
        body,
        out_shape=jax.ShapeDtypeStruct((1, 1), jnp.float32),
        out_specs=pl.BlockSpec(memory_space=pltpu.SMEM),
    )(x2)
    return out[0, 0]


def kernel(inputs, labels, num_sampled, input_embed, out_embed, noise_idx):
    inputs_f = inputs.reshape(-1).astype(jnp.int32)
    labels_f = labels.reshape(-1).astype(jnp.int32)
    noise_f = noise_idx.reshape(-1).astype(jnp.int32)
    scores = _sc_scores(inputs_f, labels_f, noise_f, input_embed, out_embed)
    return _tc_loss(scores)
